# initial kernel scaffold (unmeasured)
import jax
import jax.numpy as jnp
from jax import lax
from jax.experimental import pallas as pl
from jax.experimental.pallas import tpu as pltpu

N_DEV = 4


def kernel(x, w_mat, scale_x, scale_w):
    m_total, _ = x.shape
    n = w_mat.shape[1]
    m_per = m_total // N_DEV

    def body(x_ref, w_ref, sx_ref, sw_ref, out_ref,
             send_ref, recv_ref, send_sems, recv_sems):
        my = lax.axis_index("i")
        left = lax.rem(my + N_DEV - 1, N_DEV)
        right = lax.rem(my + 1, N_DEV)

        barrier_sem = pltpu.get_barrier_semaphore()
        for nbr in (left, right):
            pl.semaphore_signal(barrier_sem, inc=1, device_id=(nbr,),
                                device_id_type=pl.DeviceIdType.MESH)
        pl.semaphore_wait(barrier_sem, 2)

        def partial(c):
            xs = x_ref[pl.ds(c * m_per, m_per), :]
            return lax.dot_general(
                xs, w_ref[...],
                dimension_numbers=(((1,), (0,)), ((), ())),
                preferred_element_type=jnp.float32,
            )

        send_ref[...] = partial(lax.rem(my + N_DEV - 1, N_DEV))
        for s in range(N_DEV - 1):
            rdma = pltpu.make_async_remote_copy(
                src_ref=send_ref,
                dst_ref=recv_ref.at[s],
                send_sem=send_sems.at[s],
                recv_sem=recv_sems.at[s],
                device_id=(right,),
                device_id_type=pl.DeviceIdType.MESH,
            )
            rdma.start()
            rdma.wait()
            if s < N_DEV - 2:
                send_ref[...] = recv_ref[s] + partial(
                    lax.rem(my + N_DEV - 2 - s, N_DEV))

        acc = recv_ref[N_DEV - 2] + partial(my)
        y = acc * (sx_ref[0] * sw_ref[0])
        out_ref[...] = y * (1.0 / (1.0 + jnp.exp(-y)))

    return pl.pallas_call(
        body,
        out_shape=jax.ShapeDtypeStruct((m_per, n), jnp.float32),
        in_specs=[
            pl.BlockSpec(memory_space=pltpu.VMEM),
            pl.BlockSpec(memory_space=pltpu.VMEM),
            pl.BlockSpec(memory_space=pltpu.SMEM),
            pl.BlockSpec(memory_space=pltpu.SMEM),
        ],
        out_specs=pl.BlockSpec(memory_space=pltpu.VMEM),
        scratch_shapes=[
            pltpu.VMEM((m_per, n), jnp.float32),
            pltpu.VMEM((N_DEV - 1, m_per, n), jnp.float32),
            pltpu.SemaphoreType.DMA((N_DEV - 1,)),
            pltpu.SemaphoreType.DMA((N_DEV - 1,)),
        ],
        compiler_params=pltpu.CompilerParams(collective_id=0),
    )(x, w_mat, scale_x, scale_w)


# baseline (device time: 170591 ns/iter reference)
import jax
import jax.numpy as jnp
from jax import lax
from jax.experimental import pallas as pl
from jax.experimental.pallas import tpu as pltpu

N_DEV = 4


def kernel(x, w_mat, scale_x, scale_w):
    m_total, _ = x.shape
    n = w_mat.shape[1]
    m_per = m_total // N_DEV

    x = x.astype(jnp.float8_e4m3fn)
    w_mat = w_mat.astype(jnp.float8_e4m3fn)

    def body(x_ref, w_ref, sx_ref, sw_ref, out_ref,
             send_ref, recv_ref, send_sems, recv_sems):
        my = lax.axis_index("i")
        left = lax.rem(my + N_DEV - 1, N_DEV)
        right = lax.rem(my + 1, N_DEV)

        barrier_sem = pltpu.get_barrier_semaphore()
        for nbr in (left, right):
            pl.semaphore_signal(barrier_sem, inc=1, device_id=(nbr,),
                                device_id_type=pl.DeviceIdType.MESH)
        pl.semaphore_wait(barrier_sem, 2)

        def partial(c):
            xs = x_ref[pl.ds(c * m_per, m_per), :]
            return lax.dot_general(
                xs, w_ref[...],
                dimension_numbers=(((1,), (0,)), ((), ())),
                preferred_element_type=jnp.float32,
            )

        send_ref[...] = partial(lax.rem(my + N_DEV - 1, N_DEV)).astype(
            jnp.bfloat16)
        for s in range(N_DEV - 1):
            rdma = pltpu.make_async_remote_copy(
                src_ref=send_ref,
                dst_ref=recv_ref.at[s],
                send_sem=send_sems.at[s],
                recv_sem=recv_sems.at[s],
                device_id=(right,),
                device_id_type=pl.DeviceIdType.MESH,
            )
            rdma.start()
            rdma.wait()
            if s < N_DEV - 2:
                acc = recv_ref[s].astype(jnp.float32) + partial(
                    lax.rem(my + N_DEV - 2 - s, N_DEV))
                send_ref[...] = acc.astype(jnp.bfloat16)

        acc = recv_ref[N_DEV - 2].astype(jnp.float32) + partial(my)
        y = acc * (sx_ref[0] * sw_ref[0])
        out_ref[...] = y * (1.0 / (1.0 + jnp.exp(-y)))

    return pl.pallas_call(
        body,
        out_shape=jax.ShapeDtypeStruct((m_per, n), jnp.float32),
        in_specs=[
            pl.BlockSpec(memory_space=pltpu.VMEM),
            pl.BlockSpec(memory_space=pltpu.VMEM),
            pl.BlockSpec(memory_space=pltpu.SMEM),
            pl.BlockSpec(memory_space=pltpu.SMEM),
        ],
        out_specs=pl.BlockSpec(memory_space=pltpu.VMEM),
        scratch_shapes=[
            pltpu.VMEM((m_per, n), jnp.bfloat16),
            pltpu.VMEM((N_DEV - 1, m_per, n), jnp.bfloat16),
            pltpu.SemaphoreType.DMA((N_DEV - 1,)),
            pltpu.SemaphoreType.DMA((N_DEV - 1,)),
        ],
        compiler_params=pltpu.CompilerParams(collective_id=0),
    )(x, w_mat, scale_x, scale_w)


# device time: 98097 ns/iter; 1.7390x vs baseline; 1.7390x over previous
import jax
import jax.numpy as jnp
from jax import lax
from jax.experimental import pallas as pl
from jax.experimental.pallas import tpu as pltpu

N_DEV = 4


def kernel(x, w_mat, scale_x, scale_w):
    m_total, _ = x.shape
    n = w_mat.shape[1]
    m_per = m_total // N_DEV
    n_half = n // 2

    x = x.astype(jnp.float8_e4m3fn)
    w_mat = w_mat.astype(jnp.float8_e4m3fn)

    def body(x_ref, w_ref, sx_ref, sw_ref, out_ref,
             send_r_ref, send_l_ref, recv_r_ref, recv_l_ref,
             send_sems_r, recv_sems_r, send_sems_l, recv_sems_l):
        my = lax.axis_index("i")
        left = lax.rem(my + N_DEV - 1, N_DEV)
        right = lax.rem(my + 1, N_DEV)

        barrier_sem = pltpu.get_barrier_semaphore()
        for nbr in (left, right):
            pl.semaphore_signal(barrier_sem, inc=1, device_id=(nbr,),
                                device_id_type=pl.DeviceIdType.MESH)
        pl.semaphore_wait(barrier_sem, 2)

        def partial(c, lo):
            xs = x_ref[pl.ds(c * m_per, m_per), :]
            ws = w_ref[:, 0:n_half] if lo else w_ref[:, n_half:n]
            return lax.dot_general(
                xs, ws,
                dimension_numbers=(((1,), (0,)), ((), ())),
                preferred_element_type=jnp.float32,
            )

        def silu_store(acc, lo):
            y = acc * (sx_ref[0] * sw_ref[0])
            o = y * (1.0 / (1.0 + jnp.exp(-y)))
            if lo:
                out_ref[:, 0:n_half] = o
            else:
                out_ref[:, n_half:n] = o

        send_r_ref[...] = partial(lax.rem(my + N_DEV - 1, N_DEV), True
                                  ).astype(jnp.bfloat16)
        send_l_ref[...] = partial(lax.rem(my + 1, N_DEV), False
                                  ).astype(jnp.bfloat16)
        for s in range(N_DEV - 1):
            rdma_r = pltpu.make_async_remote_copy(
                src_ref=send_r_ref,
                dst_ref=recv_r_ref.at[s],
                send_sem=send_sems_r.at[s],
                recv_sem=recv_sems_r.at[s],
                device_id=(right,),
                device_id_type=pl.DeviceIdType.MESH,
            )
            rdma_l = pltpu.make_async_remote_copy(
                src_ref=send_l_ref,
                dst_ref=recv_l_ref.at[s],
                send_sem=send_sems_l.at[s],
                recv_sem=recv_sems_l.at[s],
                device_id=(left,),
                device_id_type=pl.DeviceIdType.MESH,
            )
            rdma_r.start()
            rdma_l.start()
            if s < N_DEV - 2:
                nxt_r = partial(lax.rem(my + N_DEV - 2 - s, N_DEV), True)
                nxt_l = partial(lax.rem(my + 2 + s, N_DEV), False)
                rdma_r.wait()
                send_r_ref[...] = (recv_r_ref[s].astype(jnp.float32)
                                   + nxt_r).astype(jnp.bfloat16)
                rdma_l.wait()
                send_l_ref[...] = (recv_l_ref[s].astype(jnp.float32)
                                   + nxt_l).astype(jnp.bfloat16)
            else:
                own_r = partial(my, True)
                own_l = partial(my, False)
                rdma_r.wait()
                silu_store(recv_r_ref[s].astype(jnp.float32) + own_r, True)
                rdma_l.wait()
                silu_store(recv_l_ref[s].astype(jnp.float32) + own_l, False)

    return pl.pallas_call(
        body,
        out_shape=jax.ShapeDtypeStruct((m_per, n), jnp.float32),
        in_specs=[
            pl.BlockSpec(memory_space=pltpu.VMEM),
            pl.BlockSpec(memory_space=pltpu.VMEM),
            pl.BlockSpec(memory_space=pltpu.SMEM),
            pl.BlockSpec(memory_space=pltpu.SMEM),
        ],
        out_specs=pl.BlockSpec(memory_space=pltpu.VMEM),
        scratch_shapes=[
            pltpu.VMEM((m_per, n_half), jnp.bfloat16),
            pltpu.VMEM((m_per, n_half), jnp.bfloat16),
            pltpu.VMEM((N_DEV - 1, m_per, n_half), jnp.bfloat16),
            pltpu.VMEM((N_DEV - 1, m_per, n_half), jnp.bfloat16),
            pltpu.SemaphoreType.DMA((N_DEV - 1,)),
            pltpu.SemaphoreType.DMA((N_DEV - 1,)),
            pltpu.SemaphoreType.DMA((N_DEV - 1,)),
            pltpu.SemaphoreType.DMA((N_DEV - 1,)),
        ],
        compiler_params=pltpu.CompilerParams(collective_id=0),
    )(x, w_mat, scale_x, scale_w)
